# baseline (speedup 1.0000x reference)
import jax
import jax.numpy as jnp
from jax.experimental import pallas as pl
from jax.experimental.pallas import tpu as pltpu

N = 16384
H = 256
HH = 128
N_MOL = 16
N_TYPES = 100
BN = 4096
NB = N // BN


def _fused_body(z_ref, pos_ref, batch_ref, emb_ref, Wpos_ref, W1_ref, b1_ref,
                W2_ref, b2_ref, Wo1_ref, bo1_ref, Wo2_ref, bo2_ref, mean_ref, std_ref,
                out_ref):
    i = pl.program_id(0)
    zc = z_ref[...]
    seg_row = batch_ref[0, :, :]

    oh = (zc == jax.lax.broadcasted_iota(jnp.int32, (1, N_TYPES), 1)
          ).astype(jnp.float32)
    h = jnp.dot(oh, emb_ref[...], preferred_element_type=jnp.float32)
    h = h + jnp.dot(pos_ref[...], Wpos_ref[...],
                    preferred_element_type=jnp.float32)

    h = jax.nn.silu(jnp.dot(h, W1_ref[...],
                            preferred_element_type=jnp.float32) + b1_ref[...])
    h = h + jax.nn.silu(jnp.dot(h, W2_ref[...],
                                preferred_element_type=jnp.float32)
                        + b2_ref[...])
    t = jax.nn.silu(jnp.dot(h, Wo1_ref[...],
                            preferred_element_type=jnp.float32) + bo1_ref[...])
    x = jnp.dot(t, Wo2_ref[...], preferred_element_type=jnp.float32) \
        + bo2_ref[...]
    x = x * std_ref[...] + mean_ref[...]

    ohT = (jax.lax.broadcasted_iota(jnp.int32, (N_MOL, 1), 0) == seg_row
           ).astype(jnp.float32)
    part = jnp.dot(ohT, x, preferred_element_type=jnp.float32)

    @pl.when(i == 0)
    def _():
        out_ref[...] = jnp.zeros_like(out_ref)

    out_ref[...] += part


def kernel(z, pos, batch, emb, Wpos, W1, b1, W2, b2, Wo1, bo1, Wo2, bo2,
           mean, std):
    z2 = z.astype(jnp.int32).reshape(N, 1)
    batch3 = batch.astype(jnp.int32).reshape(NB, 1, BN)

    grid = (NB,)
    out = pl.pallas_call(
        _fused_body,
        grid=grid,
        in_specs=[
            pl.BlockSpec((BN, 1), lambda i: (i, 0)),
            pl.BlockSpec((BN, 3), lambda i: (i, 0)),
            pl.BlockSpec((1, 1, BN), lambda i: (i, 0, 0)),
            pl.BlockSpec((N_TYPES, H), lambda i: (0, 0)),
            pl.BlockSpec((3, H), lambda i: (0, 0)),
            pl.BlockSpec((H, H), lambda i: (0, 0)),
            pl.BlockSpec((1, H), lambda i: (0, 0)),
            pl.BlockSpec((H, H), lambda i: (0, 0)),
            pl.BlockSpec((1, H), lambda i: (0, 0)),
            pl.BlockSpec((H, HH), lambda i: (0, 0)),
            pl.BlockSpec((1, HH), lambda i: (0, 0)),
            pl.BlockSpec((HH, 1), lambda i: (0, 0)),
            pl.BlockSpec((1, 1), lambda i: (0, 0)),
            pl.BlockSpec((1, 1), lambda i: (0, 0)),
            pl.BlockSpec((1, 1), lambda i: (0, 0)),
        ],
        out_specs=pl.BlockSpec((N_MOL, 1), lambda i: (0, 0)),
        out_shape=jax.ShapeDtypeStruct((N_MOL, 1), jnp.float32),
    )(z2, pos, batch3, emb, Wpos, W1, b1.reshape(1, H), W2,
      b2.reshape(1, H), Wo1, bo1.reshape(1, HH), Wo2, bo2.reshape(1, 1),
      mean.reshape(1, 1), std.reshape(1, 1))
    return out


# speedup vs baseline: 1.1702x; 1.1702x over previous
import jax
import jax.numpy as jnp
from jax.experimental import pallas as pl
from jax.experimental.pallas import tpu as pltpu

N = 16384
H = 256
HH = 128
N_MOL = 16
N_TYPES = 100
BN = 4096
NB = N // BN


def _fused_body(z_ref, pos_ref, batch_ref, emb_ref, Wpos_ref, W1_ref, b1_ref,
                W2_ref, b2_ref, Wo1_ref, bo1_ref, Wo2_ref, bo2_ref, mean_ref, std_ref,
                out_ref):
    i = pl.program_id(0)
    z = z_ref[0, 0, :]
    seg_row = batch_ref[0, :, :]

    oh = (z[:, None] == jax.lax.broadcasted_iota(jnp.int32, (1, N_TYPES), 1)
          ).astype(jnp.float32)
    h = jnp.dot(oh, emb_ref[...], preferred_element_type=jnp.float32)
    h = h + jnp.dot(pos_ref[...], Wpos_ref[...],
                    preferred_element_type=jnp.float32)

    h = jax.nn.silu(jnp.dot(h, W1_ref[...],
                            preferred_element_type=jnp.float32) + b1_ref[...])
    h = h + jax.nn.silu(jnp.dot(h, W2_ref[...],
                                preferred_element_type=jnp.float32)
                        + b2_ref[...])
    t = jax.nn.silu(jnp.dot(h, Wo1_ref[...],
                            preferred_element_type=jnp.float32) + bo1_ref[...])
    x = jnp.dot(t, Wo2_ref[...], preferred_element_type=jnp.float32) \
        + bo2_ref[...]
    x = x * std_ref[...] + mean_ref[...]

    ohT = (jax.lax.broadcasted_iota(jnp.int32, (N_MOL, 1), 0) == seg_row
           ).astype(jnp.float32)
    part = jnp.dot(ohT, x, preferred_element_type=jnp.float32)

    @pl.when(i == 0)
    def _():
        out_ref[...] = jnp.zeros_like(out_ref)

    out_ref[...] += part


def kernel(z, pos, batch, emb, Wpos, W1, b1, W2, b2, Wo1, bo1, Wo2, bo2,
           mean, std):
    z3 = z.astype(jnp.int32).reshape(NB, 1, BN)
    batch3 = batch.astype(jnp.int32).reshape(NB, 1, BN)

    grid = (NB,)
    out = pl.pallas_call(
        _fused_body,
        grid=grid,
        in_specs=[
            pl.BlockSpec((1, 1, BN), lambda i: (i, 0, 0)),
            pl.BlockSpec((BN, 3), lambda i: (i, 0)),
            pl.BlockSpec((1, 1, BN), lambda i: (i, 0, 0)),
            pl.BlockSpec((N_TYPES, H), lambda i: (0, 0)),
            pl.BlockSpec((3, H), lambda i: (0, 0)),
            pl.BlockSpec((H, H), lambda i: (0, 0)),
            pl.BlockSpec((1, H), lambda i: (0, 0)),
            pl.BlockSpec((H, H), lambda i: (0, 0)),
            pl.BlockSpec((1, H), lambda i: (0, 0)),
            pl.BlockSpec((H, HH), lambda i: (0, 0)),
            pl.BlockSpec((1, HH), lambda i: (0, 0)),
            pl.BlockSpec((HH, 1), lambda i: (0, 0)),
            pl.BlockSpec((1, 1), lambda i: (0, 0)),
            pl.BlockSpec((1, 1), lambda i: (0, 0)),
            pl.BlockSpec((1, 1), lambda i: (0, 0)),
        ],
        out_specs=pl.BlockSpec((N_MOL, 1), lambda i: (0, 0)),
        out_shape=jax.ShapeDtypeStruct((N_MOL, 1), jnp.float32),
    )(z3, pos, batch3, emb, Wpos, W1, b1.reshape(1, H), W2,
      b2.reshape(1, H), Wo1, bo1.reshape(1, HH), Wo2, bo2.reshape(1, 1),
      mean.reshape(1, 1), std.reshape(1, 1))
    return out


# speedup vs baseline: 1.3645x; 1.1661x over previous
import jax
import jax.numpy as jnp
from jax.experimental import pallas as pl
from jax.experimental.pallas import tpu as pltpu

N = 16384
H = 256
HH = 128
N_MOL = 16
N_TYPES = 100
BN = 4096
NB = N // BN


def _fused_body(z_ref, pos_ref, batch_ref, emb_ref, Wpos_ref, W1_ref, b1_ref,
                W2_ref, b2_ref, Wo1_ref, bo1_ref, Wo2_ref, bo2_ref, mean_ref, std_ref,
                out_ref, acc_ref, cnt_ref):
    i = pl.program_id(0)
    z = z_ref[0, 0, :]
    seg = batch_ref[0, 0, :]

    oh = (z[:, None] == jax.lax.broadcasted_iota(jnp.int32, (1, 128), 1)
          ).astype(jnp.float32)
    A = oh + jnp.pad(pos_ref[...], ((0, 0), (104, 21)))
    B = jnp.pad(emb_ref[...], ((0, 28), (0, 0))) \
        + jnp.pad(Wpos_ref[...], ((104, 21), (0, 0)))
    h = jnp.dot(A, B, preferred_element_type=jnp.float32)

    h = jax.nn.silu(jnp.dot(h, W1_ref[...],
                            preferred_element_type=jnp.float32) + b1_ref[...])
    h = h + jax.nn.silu(jnp.dot(h, W2_ref[...],
                                preferred_element_type=jnp.float32)
                        + b2_ref[...])
    t = jax.nn.silu(jnp.dot(h, Wo1_ref[...],
                            preferred_element_type=jnp.float32) + bo1_ref[...])

    ohb = (seg[:, None] == jax.lax.broadcasted_iota(jnp.int32, (1, N_MOL), 1)
           ).astype(jnp.float32)
    tseg = jax.lax.dot_general(t, ohb, (((0,), (0,)), ((), ())),
                               preferred_element_type=jnp.float32)
    cpart = jnp.sum(ohb, axis=0, keepdims=True)

    @pl.when(i == 0)
    def _():
        acc_ref[...] = jnp.zeros_like(acc_ref)
        cnt_ref[...] = jnp.zeros_like(cnt_ref)

    acc_ref[...] += tseg
    cnt_ref[...] += cpart

    @pl.when(i == NB - 1)
    def _():
        std = std_ref[...]
        per_atom_const = bo2_ref[...] * std + mean_ref[...]
        out_ref[...] = (jnp.sum(acc_ref[...] * Wo2_ref[...], axis=0,
                                keepdims=True) * std
                        + per_atom_const * cnt_ref[...])


def kernel(z, pos, batch, emb, Wpos, W1, b1, W2, b2, Wo1, bo1, Wo2, bo2,
           mean, std):
    z3 = z.astype(jnp.int32).reshape(NB, 1, BN)
    batch3 = batch.astype(jnp.int32).reshape(NB, 1, BN)

    grid = (NB,)
    out = pl.pallas_call(
        _fused_body,
        grid=grid,
        in_specs=[
            pl.BlockSpec((1, 1, BN), lambda i: (i, 0, 0)),
            pl.BlockSpec((BN, 3), lambda i: (i, 0)),
            pl.BlockSpec((1, 1, BN), lambda i: (i, 0, 0)),
            pl.BlockSpec((N_TYPES, H), lambda i: (0, 0)),
            pl.BlockSpec((3, H), lambda i: (0, 0)),
            pl.BlockSpec((H, H), lambda i: (0, 0)),
            pl.BlockSpec((1, H), lambda i: (0, 0)),
            pl.BlockSpec((H, H), lambda i: (0, 0)),
            pl.BlockSpec((1, H), lambda i: (0, 0)),
            pl.BlockSpec((H, HH), lambda i: (0, 0)),
            pl.BlockSpec((1, HH), lambda i: (0, 0)),
            pl.BlockSpec((HH, 1), lambda i: (0, 0)),
            pl.BlockSpec((1, 1), lambda i: (0, 0)),
            pl.BlockSpec((1, 1), lambda i: (0, 0)),
            pl.BlockSpec((1, 1), lambda i: (0, 0)),
        ],
        out_specs=pl.BlockSpec((1, N_MOL), lambda i: (0, 0)),
        out_shape=jax.ShapeDtypeStruct((1, N_MOL), jnp.float32),
        scratch_shapes=[pltpu.VMEM((HH, N_MOL), jnp.float32),
                        pltpu.VMEM((1, N_MOL), jnp.float32)],
    )(z3, pos, batch3, emb, Wpos, W1, b1.reshape(1, H), W2,
      b2.reshape(1, H), Wo1, bo1.reshape(1, HH), Wo2, bo2.reshape(1, 1),
      mean.reshape(1, 1), std.reshape(1, 1))
    return out.reshape(N_MOL, 1)


# speedup vs baseline: 1.4730x; 1.0795x over previous
import jax
import jax.numpy as jnp


def _silu(v):
    u = 0.5 * v
    return u * jnp.tanh(u) + u
from jax.experimental import pallas as pl
from jax.experimental.pallas import tpu as pltpu

N = 16384
H = 256
HH = 128
N_MOL = 16
N_TYPES = 100
BN = 4096
NB = N // BN


def _fused_body(z_ref, pos_ref, batch_ref, emb_ref, Wpos_ref, W1_ref, b1_ref,
                W2_ref, b2_ref, Wo1_ref, bo1_ref, Wo2_ref, bo2_ref, mean_ref, std_ref,
                out_ref, acc_ref, cnt_ref):
    i = pl.program_id(0)
    z = z_ref[0, 0, :]
    seg = batch_ref[0, 0, :]

    oh = (z[:, None] == jax.lax.broadcasted_iota(jnp.int32, (1, 128), 1)
          ).astype(jnp.float32)
    A = oh + jnp.pad(pos_ref[...], ((0, 0), (104, 21)))
    B = jnp.pad(emb_ref[...], ((0, 28), (0, 0))) \
        + jnp.pad(Wpos_ref[...], ((104, 21), (0, 0)))
    h = jnp.dot(A, B, preferred_element_type=jnp.float32)

    h = _silu(jnp.dot(h, W1_ref[...],
                            preferred_element_type=jnp.float32) + b1_ref[...])
    h = h + _silu(jnp.dot(h, W2_ref[...],
                                preferred_element_type=jnp.float32)
                        + b2_ref[...])
    t = _silu(jnp.dot(h, Wo1_ref[...],
                            preferred_element_type=jnp.float32) + bo1_ref[...])

    ohb = (seg[:, None] == jax.lax.broadcasted_iota(jnp.int32, (1, N_MOL), 1)
           ).astype(jnp.float32)
    tseg = jax.lax.dot_general(t, ohb, (((0,), (0,)), ((), ())),
                               preferred_element_type=jnp.float32)
    cpart = jnp.sum(ohb, axis=0, keepdims=True)

    @pl.when(i == 0)
    def _():
        acc_ref[...] = jnp.zeros_like(acc_ref)
        cnt_ref[...] = jnp.zeros_like(cnt_ref)

    acc_ref[...] += tseg
    cnt_ref[...] += cpart

    @pl.when(i == NB - 1)
    def _():
        std = std_ref[...]
        per_atom_const = bo2_ref[...] * std + mean_ref[...]
        out_ref[...] = (jnp.sum(acc_ref[...] * Wo2_ref[...], axis=0,
                                keepdims=True) * std
                        + per_atom_const * cnt_ref[...])


def kernel(z, pos, batch, emb, Wpos, W1, b1, W2, b2, Wo1, bo1, Wo2, bo2,
           mean, std):
    z3 = z.astype(jnp.int32).reshape(NB, 1, BN)
    batch3 = batch.astype(jnp.int32).reshape(NB, 1, BN)

    grid = (NB,)
    out = pl.pallas_call(
        _fused_body,
        grid=grid,
        in_specs=[
            pl.BlockSpec((1, 1, BN), lambda i: (i, 0, 0)),
            pl.BlockSpec((BN, 3), lambda i: (i, 0)),
            pl.BlockSpec((1, 1, BN), lambda i: (i, 0, 0)),
            pl.BlockSpec((N_TYPES, H), lambda i: (0, 0)),
            pl.BlockSpec((3, H), lambda i: (0, 0)),
            pl.BlockSpec((H, H), lambda i: (0, 0)),
            pl.BlockSpec((1, H), lambda i: (0, 0)),
            pl.BlockSpec((H, H), lambda i: (0, 0)),
            pl.BlockSpec((1, H), lambda i: (0, 0)),
            pl.BlockSpec((H, HH), lambda i: (0, 0)),
            pl.BlockSpec((1, HH), lambda i: (0, 0)),
            pl.BlockSpec((HH, 1), lambda i: (0, 0)),
            pl.BlockSpec((1, 1), lambda i: (0, 0)),
            pl.BlockSpec((1, 1), lambda i: (0, 0)),
            pl.BlockSpec((1, 1), lambda i: (0, 0)),
        ],
        out_specs=pl.BlockSpec((1, N_MOL), lambda i: (0, 0)),
        out_shape=jax.ShapeDtypeStruct((1, N_MOL), jnp.float32),
        scratch_shapes=[pltpu.VMEM((HH, N_MOL), jnp.float32),
                        pltpu.VMEM((1, N_MOL), jnp.float32)],
    )(z3, pos, batch3, emb, Wpos, W1, b1.reshape(1, H), W2,
      b2.reshape(1, H), Wo1, bo1.reshape(1, HH), Wo2, bo2.reshape(1, 1),
      mean.reshape(1, 1), std.reshape(1, 1))
    return out.reshape(N_MOL, 1)
